# Initial kernel scaffold; baseline (speedup 1.0000x reference)
import jax
import jax.numpy as jnp
from jax.experimental import pallas as pl


def kernel(x, edge_index, batch, Wl1, bl1, Wr1, g1, b1, Wl2, bl2, Wr2, Wa, ba, gn1, bn1, Wb, bb, gn2, bn2, Wc, bc, gn3, bn3, Wx, bx, Wy, by, Wrot, brot):
    raise NotImplementedError("write your pallas kernel here")


# speedup vs baseline: 12.2323x; 12.2323x over previous
import functools

import jax
import jax.numpy as jnp
from jax import lax
from jax.experimental import pallas as pl
from jax.experimental.pallas import tpu as pltpu
from jax.experimental.pallas import tpu_sc as plsc

N = 10000
E = 320000
D = 128
H = 16
B = 64
NC = 2
NS = 16
NW = NC * NS
NPAD = 10240
EW = E // NW
C = 128
K = -(-EW // C)
EWP = K * C
SLAB = NPAD // NS

F32 = jnp.float32


def _proj_body(x_ref, wl_ref, wr_ref, p_ref, q_ref):
    x = x_ref[...]
    p_ref[...] = jnp.dot(x, wl_ref[...], preferred_element_type=F32)
    q_ref[...] = jnp.dot(x, wr_ref[...], preferred_element_type=F32)


def _proj(x, WlT, WrT):
    G = 10
    return pl.pallas_call(
        _proj_body,
        grid=(G,),
        in_specs=[
            pl.BlockSpec((N // G, D), lambda i: (i, 0)),
            pl.BlockSpec((D, H), lambda i: (0, 0)),
            pl.BlockSpec((D, H), lambda i: (0, 0)),
        ],
        out_specs=[
            pl.BlockSpec((N // G, H), lambda i: (i, 0)),
            pl.BlockSpec((N // G, H), lambda i: (i, 0)),
        ],
        out_shape=[jax.ShapeDtypeStruct((N, H), F32)] * 2,
    )(x, WlT, WrT)


def _sc_edges_count_body(table, srcp, dstp, zrows, zcnt, ones,
                         acc_out, cnt_out,
                         src_v, dst_v, rows_v, ones_v, acc_s, cnt_s, sem):
    cid = lax.axis_index("c")
    sid = lax.axis_index("s")
    w = sid * NC + cid
    base = sid * SLAB

    pltpu.sync_copy(zrows, acc_s.at[pl.ds(base, SLAB)])
    pltpu.sync_copy(zcnt, cnt_s.at[pl.ds(base, SLAB)])
    pltpu.sync_copy(ones, ones_v)
    pltpu.sync_copy(srcp.at[w], src_v)
    pltpu.sync_copy(dstp.at[w], dst_v)
    plsc.subcore_barrier()

    def chunk(j, carry):
        pltpu.async_copy(table.at[src_v.at[j]], rows_v, sem).wait()
        pltpu.sync_copy(rows_v, acc_s.at[dst_v.at[j]], add=True)
        pltpu.sync_copy(ones_v, cnt_s.at[dst_v.at[j]], add=True)
        return carry

    lax.fori_loop(0, K, chunk, 0)
    plsc.subcore_barrier()

    pltpu.sync_copy(acc_s.at[pl.ds(base, SLAB)],
                    acc_out.at[cid, pl.ds(base, SLAB)])
    pltpu.sync_copy(cnt_s.at[pl.ds(base, SLAB)],
                    cnt_out.at[cid, pl.ds(base, SLAB)])


def _sc_edges_body(table, srcp, dstp, zrows,
                   acc_out,
                   src_v, dst_v, rows_v, acc_s, sem):
    cid = lax.axis_index("c")
    sid = lax.axis_index("s")
    w = sid * NC + cid
    base = sid * SLAB

    pltpu.sync_copy(zrows, acc_s.at[pl.ds(base, SLAB)])
    pltpu.sync_copy(srcp.at[w], src_v)
    pltpu.sync_copy(dstp.at[w], dst_v)
    plsc.subcore_barrier()

    def chunk(j, carry):
        pltpu.async_copy(table.at[src_v.at[j]], rows_v, sem).wait()
        pltpu.sync_copy(rows_v, acc_s.at[dst_v.at[j]], add=True)
        return carry

    lax.fori_loop(0, K, chunk, 0)
    plsc.subcore_barrier()

    pltpu.sync_copy(acc_s.at[pl.ds(base, SLAB)],
                    acc_out.at[cid, pl.ds(base, SLAB)])


_SC_MESH = plsc.VectorSubcoreMesh(core_axis_name="c", subcore_axis_name="s")
_SC_PARAMS = pltpu.CompilerParams(use_tc_tiling_on_sc=False)


def _sc_edges_count(table, srcp, dstp, zrows, zcnt, ones):
    fn = pl.kernel(
        _sc_edges_count_body,
        out_type=[
            jax.ShapeDtypeStruct((NC, NPAD, H), F32),
            jax.ShapeDtypeStruct((NC, NPAD), F32),
        ],
        mesh=_SC_MESH,
        scratch_types=[
            pltpu.VMEM((K, C), jnp.int32),
            pltpu.VMEM((K, C), jnp.int32),
            pltpu.VMEM((C, H), F32),
            pltpu.VMEM((C,), F32),
            pltpu.VMEM_SHARED((NPAD, H), F32),
            pltpu.VMEM_SHARED((NPAD,), F32),
            pltpu.SemaphoreType.DMA,
        ],
        compiler_params=_SC_PARAMS,
    )
    return fn(table, srcp, dstp, zrows, zcnt, ones)


def _sc_edges(table, srcp, dstp, zrows):
    fn = pl.kernel(
        _sc_edges_body,
        out_type=jax.ShapeDtypeStruct((NC, NPAD, H), F32),
        mesh=_SC_MESH,
        scratch_types=[
            pltpu.VMEM((K, C), jnp.int32),
            pltpu.VMEM((K, C), jnp.int32),
            pltpu.VMEM((C, H), F32),
            pltpu.VMEM_SHARED((NPAD, H), F32),
            pltpu.SemaphoreType.DMA,
        ],
        compiler_params=_SC_PARAMS,
    )
    return fn(table, srcp, dstp, zrows)


def _mid_body(acc_ref, cnt_ref, q_ref, bl1_ref, g1_ref, b1_ref,
              wl2_ref, wr2_ref, bl2_ref, p2_ref, q2_ref):
    acc = (acc_ref[0] + acc_ref[1])[:N]
    cnt = (cnt_ref[0] + cnt_ref[1])[:N]
    mean = acc / jnp.maximum(cnt, 1.0)
    h = jax.nn.relu(mean + bl1_ref[...] + q_ref[...])
    m = jnp.mean(h, axis=0, keepdims=True)
    v = jnp.mean((h - m) * (h - m), axis=0, keepdims=True)
    h = (h - m) * jax.lax.rsqrt(v + 1e-5) * g1_ref[...] + b1_ref[...]
    p2_ref[...] = jnp.dot(h, wl2_ref[...], preferred_element_type=F32)
    q2_ref[...] = jnp.dot(h, wr2_ref[...], preferred_element_type=F32) \
        + bl2_ref[...]


def _mid(acc, cnt, q1, bl1, g1, b1, Wl2T, Wr2T, bl2):
    return pl.pallas_call(
        _mid_body,
        out_shape=[jax.ShapeDtypeStruct((N, H), F32)] * 2,
    )(acc, cnt, q1, bl1, g1, b1, Wl2T, Wr2T, bl2)


def _bn_rows(x, g, b):
    m = jnp.mean(x, axis=0, keepdims=True)
    v = jnp.mean((x - m) * (x - m), axis=0, keepdims=True)
    return (x - m) * jax.lax.rsqrt(v + 1e-5) * g + b


def _softmax(z):
    z = z - jnp.max(z, axis=1, keepdims=True)
    e = jnp.exp(z)
    return e / jnp.sum(e, axis=1, keepdims=True)


def _head_body(acc_ref, cnt_ref, q2_ref, batch_ref,
               waT_ref, ba_ref, gn1_ref, bn1_ref,
               wbT_ref, bb_ref, gn2_ref, bn2_ref,
               wcT_ref, bc_ref, gn3_ref, bn3_ref,
               wxT_ref, bx_ref, wyT_ref, by_ref, wrotT_ref, brot_ref,
               xx_ref, y_ref, rot_ref):
    acc = (acc_ref[0] + acc_ref[1])[:N]
    cnt = (cnt_ref[0] + cnt_ref[1])[:N]
    mean = acc / jnp.maximum(cnt, 1.0)
    h2 = jax.nn.relu(mean + q2_ref[...])

    onehot = (batch_ref[...] ==
              lax.broadcasted_iota(jnp.int32, (1, B), 1)).astype(F32)
    h2e = jnp.concatenate([h2, jnp.ones((N, 1), F32)], axis=1)
    sums = lax.dot_general(onehot, h2e, (((0,), (0,)), ((), ())),
                           preferred_element_type=F32)
    x_enc = sums[:, :H] / jnp.maximum(sums[:, H:H + 1], 1.0)

    t = _bn_rows(x_enc, gn1_ref[...], bn1_ref[...])
    t = jax.nn.relu(jnp.dot(t, waT_ref[...], preferred_element_type=F32)
                    + ba_ref[...])
    comb = jnp.concatenate([t, x_enc], axis=1)
    t = _bn_rows(comb, gn2_ref[...], bn2_ref[...])
    t = jax.nn.relu(jnp.dot(t, wbT_ref[...], preferred_element_type=F32)
                    + bb_ref[...])
    comb = jnp.concatenate([t, x_enc], axis=1)
    t = _bn_rows(comb, gn3_ref[...], bn3_ref[...])
    t = jax.nn.relu(jnp.dot(t, wcT_ref[...], preferred_element_type=F32)
                    + bc_ref[...])
    xx_ref[...] = _softmax(jnp.dot(t, wxT_ref[...],
                                   preferred_element_type=F32) + bx_ref[...])
    y_ref[...] = _softmax(jnp.dot(t, wyT_ref[...],
                                  preferred_element_type=F32) + by_ref[...])
    rot_ref[...] = _softmax(jnp.dot(t, wrotT_ref[...],
                                    preferred_element_type=F32) + brot_ref[...])


def _head(acc, cnt, q2, batch2d, args):
    return pl.pallas_call(
        _head_body,
        out_shape=[
            jax.ShapeDtypeStruct((B, 16), F32),
            jax.ShapeDtypeStruct((B, 16), F32),
            jax.ShapeDtypeStruct((B, 4), F32),
        ],
    )(acc, cnt, q2, batch2d, *args)


def kernel(x, edge_index, batch, Wl1, bl1, Wr1, g1, b1, Wl2, bl2, Wr2,
           Wa, ba, gn1, bn1, Wb, bb, gn2, bn2, Wc, bc, gn3, bn3,
           Wx, bx, Wy, by, Wrot, brot):
    row = lambda a: a.reshape(1, -1)

    src = edge_index[0].reshape(NW, EW)
    dst = edge_index[1].reshape(NW, EW)
    srcp = jnp.pad(src, ((0, 0), (0, EWP - EW))).reshape(NW, K, C)
    dstp = jnp.pad(dst, ((0, 0), (0, EWP - EW)),
                   constant_values=N).reshape(NW, K, C)
    zrows = jnp.zeros((SLAB, H), F32)
    zcnt = jnp.zeros((SLAB,), F32)
    ones = jnp.ones((C,), F32)

    p1, q1 = _proj(x, Wl1.T, Wr1.T)
    acc1, cnt = _sc_edges_count(p1, srcp, dstp, zrows, zcnt, ones)
    cnt3 = cnt[..., None]
    p2, q2 = _mid(acc1, cnt3, q1, row(bl1), row(g1), row(b1),
                  Wl2.T, Wr2.T, row(bl2))
    acc2 = _sc_edges(p2, srcp, dstp, zrows)
    head_args = (Wa.T, row(ba), row(gn1), row(bn1),
                 Wb.T, row(bb), row(gn2), row(bn2),
                 Wc.T, row(bc), row(gn3), row(bn3),
                 Wx.T, row(bx), Wy.T, row(by), Wrot.T, row(brot))
    return _head(acc2, cnt3, q2, batch.reshape(N, 1), head_args)


# speedup vs baseline: 13.4405x; 1.0988x over previous
import functools

import jax
import jax.numpy as jnp
from jax import lax
from jax.experimental import pallas as pl
from jax.experimental.pallas import tpu as pltpu
from jax.experimental.pallas import tpu_sc as plsc

N = 10000
E = 320000
D = 128
H = 16
B = 64
NC = 2
NS = 16
NW = NC * NS
NPAD = 10240
EW = E // NW
C = 128
K = 2 * (-(-EW // (2 * C)))
EWP = K * C
SLAB = NPAD // NS

F32 = jnp.float32


def _proj_body(x_ref, wl_ref, wr_ref, p_ref, q_ref):
    x = x_ref[...]
    p_ref[...] = jnp.dot(x, wl_ref[...], preferred_element_type=F32)
    q_ref[...] = jnp.dot(x, wr_ref[...], preferred_element_type=F32)


def _proj(x, WlT, WrT):
    G = 10
    return pl.pallas_call(
        _proj_body,
        grid=(G,),
        in_specs=[
            pl.BlockSpec((N // G, D), lambda i: (i, 0)),
            pl.BlockSpec((D, H), lambda i: (0, 0)),
            pl.BlockSpec((D, H), lambda i: (0, 0)),
        ],
        out_specs=[
            pl.BlockSpec((N // G, H), lambda i: (i, 0)),
            pl.BlockSpec((N // G, H), lambda i: (i, 0)),
        ],
        out_shape=[jax.ShapeDtypeStruct((N, H), F32)] * 2,
    )(x, WlT, WrT)


def _edge_pipeline(table, src_v, dst_v, acc_s, r0_v, r1_v, sem0, sem1,
                   count_fn=None):
    pltpu.async_copy(table.at[src_v.at[0]], r0_v, sem0)
    pltpu.async_copy(table.at[src_v.at[1]], r1_v, sem1)

    def pair(t, carry):
        j0 = 2 * t
        pltpu.make_async_copy(table.at[src_v.at[0]], r0_v, sem0).wait()
        pltpu.sync_copy(r0_v, acc_s.at[dst_v.at[j0]], add=True)
        if count_fn is not None:
            count_fn(j0)

        @pl.when(j0 + 2 < K)
        def _():
            pltpu.async_copy(table.at[src_v.at[j0 + 2]], r0_v, sem0)

        j1 = j0 + 1
        pltpu.make_async_copy(table.at[src_v.at[1]], r1_v, sem1).wait()
        pltpu.sync_copy(r1_v, acc_s.at[dst_v.at[j1]], add=True)
        if count_fn is not None:
            count_fn(j1)

        @pl.when(j1 + 2 < K)
        def _():
            pltpu.async_copy(table.at[src_v.at[j1 + 2]], r1_v, sem1)

        return carry

    lax.fori_loop(0, K // 2, pair, 0)


def _sc_edges_count_body(table, srcp, dstp, zrows, zcnt, ones,
                         acc_out, cnt_out,
                         src_v, dst_v, r0_v, r1_v, ones_v, acc_s, cnt_s,
                         sem0, sem1):
    cid = lax.axis_index("c")
    sid = lax.axis_index("s")
    w = sid * NC + cid
    base = sid * SLAB

    pltpu.sync_copy(zrows, acc_s.at[pl.ds(base, SLAB)])
    pltpu.sync_copy(zcnt, cnt_s.at[pl.ds(base, SLAB)])
    pltpu.sync_copy(ones, ones_v)
    pltpu.sync_copy(srcp.at[w], src_v)
    pltpu.sync_copy(dstp.at[w], dst_v)
    plsc.subcore_barrier()

    def count(j):
        pltpu.sync_copy(ones_v, cnt_s.at[dst_v.at[j]], add=True)

    _edge_pipeline(table, src_v, dst_v, acc_s, r0_v, r1_v, sem0, sem1, count)
    plsc.subcore_barrier()

    pltpu.sync_copy(acc_s.at[pl.ds(base, SLAB)],
                    acc_out.at[cid, pl.ds(base, SLAB)])
    pltpu.sync_copy(cnt_s.at[pl.ds(base, SLAB)],
                    cnt_out.at[cid, pl.ds(base, SLAB)])


def _sc_edges_body(table, srcp, dstp, zrows,
                   acc_out,
                   src_v, dst_v, r0_v, r1_v, acc_s, sem0, sem1):
    cid = lax.axis_index("c")
    sid = lax.axis_index("s")
    w = sid * NC + cid
    base = sid * SLAB

    pltpu.sync_copy(zrows, acc_s.at[pl.ds(base, SLAB)])
    pltpu.sync_copy(srcp.at[w], src_v)
    pltpu.sync_copy(dstp.at[w], dst_v)
    plsc.subcore_barrier()

    _edge_pipeline(table, src_v, dst_v, acc_s, r0_v, r1_v, sem0, sem1)
    plsc.subcore_barrier()

    pltpu.sync_copy(acc_s.at[pl.ds(base, SLAB)],
                    acc_out.at[cid, pl.ds(base, SLAB)])


_SC_MESH = plsc.VectorSubcoreMesh(core_axis_name="c", subcore_axis_name="s")
_SC_PARAMS = pltpu.CompilerParams(use_tc_tiling_on_sc=False)


def _sc_edges_count(table, srcp, dstp, zrows, zcnt, ones):
    fn = pl.kernel(
        _sc_edges_count_body,
        out_type=[
            jax.ShapeDtypeStruct((NC, NPAD, H), F32),
            jax.ShapeDtypeStruct((NC, NPAD), F32),
        ],
        mesh=_SC_MESH,
        scratch_types=[
            pltpu.VMEM((K, C), jnp.int32),
            pltpu.VMEM((K, C), jnp.int32),
            pltpu.VMEM((C, H), F32),
            pltpu.VMEM((C, H), F32),
            pltpu.VMEM((C,), F32),
            pltpu.VMEM_SHARED((NPAD, H), F32),
            pltpu.VMEM_SHARED((NPAD,), F32),
            pltpu.SemaphoreType.DMA,
            pltpu.SemaphoreType.DMA,
        ],
        compiler_params=_SC_PARAMS,
    )
    return fn(table, srcp, dstp, zrows, zcnt, ones)


def _sc_edges(table, srcp, dstp, zrows):
    fn = pl.kernel(
        _sc_edges_body,
        out_type=jax.ShapeDtypeStruct((NC, NPAD, H), F32),
        mesh=_SC_MESH,
        scratch_types=[
            pltpu.VMEM((K, C), jnp.int32),
            pltpu.VMEM((K, C), jnp.int32),
            pltpu.VMEM((C, H), F32),
            pltpu.VMEM((C, H), F32),
            pltpu.VMEM_SHARED((NPAD, H), F32),
            pltpu.SemaphoreType.DMA,
            pltpu.SemaphoreType.DMA,
        ],
        compiler_params=_SC_PARAMS,
    )
    return fn(table, srcp, dstp, zrows)


def _mid_body(acc_ref, cnt_ref, q_ref, bl1_ref, g1_ref, b1_ref,
              wl2_ref, wr2_ref, bl2_ref, p2_ref, q2_ref):
    acc = (acc_ref[0] + acc_ref[1])[:N]
    cnt = (cnt_ref[0] + cnt_ref[1])[:N]
    mean = acc / jnp.maximum(cnt, 1.0)
    h = jax.nn.relu(mean + bl1_ref[...] + q_ref[...])
    m = jnp.mean(h, axis=0, keepdims=True)
    v = jnp.mean((h - m) * (h - m), axis=0, keepdims=True)
    h = (h - m) * jax.lax.rsqrt(v + 1e-5) * g1_ref[...] + b1_ref[...]
    p2_ref[...] = jnp.dot(h, wl2_ref[...], preferred_element_type=F32)
    q2_ref[...] = jnp.dot(h, wr2_ref[...], preferred_element_type=F32) \
        + bl2_ref[...]


def _mid(acc, cnt, q1, bl1, g1, b1, Wl2T, Wr2T, bl2):
    return pl.pallas_call(
        _mid_body,
        out_shape=[jax.ShapeDtypeStruct((N, H), F32)] * 2,
    )(acc, cnt, q1, bl1, g1, b1, Wl2T, Wr2T, bl2)


def _bn_rows(x, g, b):
    m = jnp.mean(x, axis=0, keepdims=True)
    v = jnp.mean((x - m) * (x - m), axis=0, keepdims=True)
    return (x - m) * jax.lax.rsqrt(v + 1e-5) * g + b


def _softmax(z):
    z = z - jnp.max(z, axis=1, keepdims=True)
    e = jnp.exp(z)
    return e / jnp.sum(e, axis=1, keepdims=True)


def _head_body(acc_ref, cnt_ref, q2_ref, batch_ref,
               waT_ref, ba_ref, gn1_ref, bn1_ref,
               wbT_ref, bb_ref, gn2_ref, bn2_ref,
               wcT_ref, bc_ref, gn3_ref, bn3_ref,
               wxT_ref, bx_ref, wyT_ref, by_ref, wrotT_ref, brot_ref,
               xx_ref, y_ref, rot_ref):
    acc = (acc_ref[0] + acc_ref[1])[:N]
    cnt = (cnt_ref[0] + cnt_ref[1])[:N]
    mean = acc / jnp.maximum(cnt, 1.0)
    h2 = jax.nn.relu(mean + q2_ref[...])

    onehot = (batch_ref[...] ==
              lax.broadcasted_iota(jnp.int32, (1, B), 1)).astype(F32)
    h2e = jnp.concatenate([h2, jnp.ones((N, 1), F32)], axis=1)
    sums = lax.dot_general(onehot, h2e, (((0,), (0,)), ((), ())),
                           preferred_element_type=F32)
    x_enc = sums[:, :H] / jnp.maximum(sums[:, H:H + 1], 1.0)

    t = _bn_rows(x_enc, gn1_ref[...], bn1_ref[...])
    t = jax.nn.relu(jnp.dot(t, waT_ref[...], preferred_element_type=F32)
                    + ba_ref[...])
    comb = jnp.concatenate([t, x_enc], axis=1)
    t = _bn_rows(comb, gn2_ref[...], bn2_ref[...])
    t = jax.nn.relu(jnp.dot(t, wbT_ref[...], preferred_element_type=F32)
                    + bb_ref[...])
    comb = jnp.concatenate([t, x_enc], axis=1)
    t = _bn_rows(comb, gn3_ref[...], bn3_ref[...])
    t = jax.nn.relu(jnp.dot(t, wcT_ref[...], preferred_element_type=F32)
                    + bc_ref[...])
    xx_ref[...] = _softmax(jnp.dot(t, wxT_ref[...],
                                   preferred_element_type=F32) + bx_ref[...])
    y_ref[...] = _softmax(jnp.dot(t, wyT_ref[...],
                                  preferred_element_type=F32) + by_ref[...])
    rot_ref[...] = _softmax(jnp.dot(t, wrotT_ref[...],
                                    preferred_element_type=F32) + brot_ref[...])


def _head(acc, cnt, q2, batch2d, args):
    return pl.pallas_call(
        _head_body,
        out_shape=[
            jax.ShapeDtypeStruct((B, 16), F32),
            jax.ShapeDtypeStruct((B, 16), F32),
            jax.ShapeDtypeStruct((B, 4), F32),
        ],
    )(acc, cnt, q2, batch2d, *args)


def kernel(x, edge_index, batch, Wl1, bl1, Wr1, g1, b1, Wl2, bl2, Wr2,
           Wa, ba, gn1, bn1, Wb, bb, gn2, bn2, Wc, bc, gn3, bn3,
           Wx, bx, Wy, by, Wrot, brot):
    row = lambda a: a.reshape(1, -1)

    src = edge_index[0].reshape(NW, EW)
    dst = edge_index[1].reshape(NW, EW)
    srcp = jnp.pad(src, ((0, 0), (0, EWP - EW))).reshape(NW, K, C)
    dstp = jnp.pad(dst, ((0, 0), (0, EWP - EW)),
                   constant_values=N).reshape(NW, K, C)
    zrows = jnp.zeros((SLAB, H), F32)
    zcnt = jnp.zeros((SLAB,), F32)
    ones = jnp.ones((C,), F32)

    p1, q1 = _proj(x, Wl1.T, Wr1.T)
    acc1, cnt = _sc_edges_count(p1, srcp, dstp, zrows, zcnt, ones)
    cnt3 = cnt[..., None]
    p2, q2 = _mid(acc1, cnt3, q1, row(bl1), row(g1), row(b1),
                  Wl2.T, Wr2.T, row(bl2))
    acc2 = _sc_edges(p2, srcp, dstp, zrows)
    head_args = (Wa.T, row(ba), row(gn1), row(bn1),
                 Wb.T, row(bb), row(gn2), row(bn2),
                 Wc.T, row(bc), row(gn3), row(bn3),
                 Wx.T, row(bx), Wy.T, row(by), Wrot.T, row(brot))
    return _head(acc2, cnt3, q2, batch.reshape(N, 1), head_args)
